# baseline (speedup 1.0000x reference)
import functools

import jax
import jax.numpy as jnp
from jax import lax
from jax.experimental import pallas as pl
from jax.experimental.pallas import tpu as pltpu, tpu_sc as plsc

NUM_GRAPHS = 512
N_NODES = 100000
DIM = 128

CHUNK = 80
NUM_CHUNKS = N_NODES // CHUNK
NUM_WORKERS = 32
ITERS = -(-NUM_CHUNKS // NUM_WORKERS)


def _mlp_body(p_ref, vn_ref, w1_ref, b1_ref, g1_ref, be1_ref,
              w2_ref, b2_ref, g2_ref, be2_ref, out_ref):
    vn = vn_ref[...] + p_ref[0] + p_ref[1]
    h = jax.lax.dot_general(
        vn, w1_ref[...], dimension_numbers=(((1,), (1,)), ((), ())),
        preferred_element_type=jnp.float32,
    ) + b1_ref[...]
    mu = jnp.mean(h, axis=0, keepdims=True)
    var = jnp.mean((h - mu) ** 2, axis=0, keepdims=True)
    h = g1_ref[...] * (h - mu) * jax.lax.rsqrt(var + 1e-5) + be1_ref[...]
    h = jnp.maximum(h, 0.0)
    h = jax.lax.dot_general(
        h, w2_ref[...], dimension_numbers=(((1,), (1,)), ((), ())),
        preferred_element_type=jnp.float32,
    ) + b2_ref[...]
    mu2 = jnp.mean(h, axis=0, keepdims=True)
    var2 = jnp.mean((h - mu2) ** 2, axis=0, keepdims=True)
    h = g2_ref[...] * (h - mu2) * jax.lax.rsqrt(var2 + 1e-5) + be2_ref[...]
    out_ref[...] = jnp.maximum(h, 0.0)


def _make_sc_segsum():
    mesh = plsc.VectorSubcoreMesh(core_axis_name="c", subcore_axis_name="s")

    @functools.partial(
        pl.kernel,
        mesh=mesh,
        out_type=jax.ShapeDtypeStruct((2, NUM_GRAPHS, DIM), jnp.float32),
        scratch_types=[
            pltpu.VMEM((CHUNK,), jnp.int32),
            pltpu.VMEM((CHUNK, DIM), jnp.float32),
            pltpu.VMEM_SHARED((NUM_GRAPHS, DIM), jnp.float32),
        ],
    )
    def segsum(emb_hbm, bv_hbm, zeros_hbm, out_hbm, idx_v, rows_v, acc_sh):
        cid = lax.axis_index("c")
        sid = lax.axis_index("s")
        wid = sid * 2 + cid

        @pl.when(sid == 0)
        def _():
            pltpu.sync_copy(zeros_hbm, acc_sh)

        plsc.subcore_barrier()

        def body(i, carry):
            c = wid + i * NUM_WORKERS

            @pl.when(c < NUM_CHUNKS)
            def _():
                base = c * CHUNK
                pltpu.sync_copy(bv_hbm.at[pl.ds(base, CHUNK)], idx_v)
                pltpu.sync_copy(emb_hbm.at[pl.ds(base, CHUNK)], rows_v)
                pltpu.sync_copy(rows_v, acc_sh.at[idx_v], add=True)

            return carry

        lax.fori_loop(0, ITERS, body, 0)
        plsc.subcore_barrier()

        @pl.when(sid == 0)
        def _():
            pltpu.sync_copy(acc_sh, out_hbm.at[cid])

    return segsum


_sc_segsum = _make_sc_segsum()


@jax.jit
def kernel(virtual_node, embeddings, batch_vector, W1, b1, g1, be1,
           W2, b2, g2, be2):
    bv = batch_vector.astype(jnp.int32)
    zeros = jnp.zeros((NUM_GRAPHS, DIM), jnp.float32)
    partials = _sc_segsum(embeddings, bv, zeros)

    full = lambda s: pl.BlockSpec(s, lambda: (0,) * len(s))
    out = pl.pallas_call(
        _mlp_body,
        in_specs=[
            full((2, NUM_GRAPHS, DIM)), full((NUM_GRAPHS, DIM)),
            full((2 * DIM, DIM)), full((1, 2 * DIM)), full((1, 2 * DIM)),
            full((1, 2 * DIM)),
            full((DIM, 2 * DIM)), full((1, DIM)), full((1, DIM)),
            full((1, DIM)),
        ],
        out_specs=full((NUM_GRAPHS, DIM)),
        out_shape=jax.ShapeDtypeStruct((NUM_GRAPHS, DIM), jnp.float32),
    )(partials, virtual_node, W1, b1.reshape(1, -1), g1.reshape(1, -1),
      be1.reshape(1, -1), W2, b2.reshape(1, -1), g2.reshape(1, -1),
      be2.reshape(1, -1))
    return out


# speedup vs baseline: 1.7866x; 1.7866x over previous
import functools

import jax
import jax.numpy as jnp
from jax import lax
from jax.experimental import pallas as pl
from jax.experimental.pallas import tpu as pltpu, tpu_sc as plsc

NUM_GRAPHS = 512
N_NODES = 100000
DIM = 128

CHUNK = 128
NUM_WORKERS = 32
CPW = 25
NBUF = 5
ACC_ROWS = NUM_GRAPHS + 8
LAST_BASE = N_NODES - CHUNK


def _mlp_body(p_ref, vn_ref, w1_ref, b1_ref, g1_ref, be1_ref,
              w2_ref, b2_ref, g2_ref, be2_ref, out_ref):
    vn = vn_ref[...] + p_ref[0] + p_ref[1]
    h = jax.lax.dot_general(
        vn, w1_ref[...], dimension_numbers=(((1,), (1,)), ((), ())),
        preferred_element_type=jnp.float32,
    ) + b1_ref[...]
    mu = jnp.mean(h, axis=0, keepdims=True)
    var = jnp.mean((h - mu) ** 2, axis=0, keepdims=True)
    h = g1_ref[...] * (h - mu) * jax.lax.rsqrt(var + 1e-5) + be1_ref[...]
    h = jnp.maximum(h, 0.0)
    h = jax.lax.dot_general(
        h, w2_ref[...], dimension_numbers=(((1,), (1,)), ((), ())),
        preferred_element_type=jnp.float32,
    ) + b2_ref[...]
    mu2 = jnp.mean(h, axis=0, keepdims=True)
    var2 = jnp.mean((h - mu2) ** 2, axis=0, keepdims=True)
    h = g2_ref[...] * (h - mu2) * jax.lax.rsqrt(var2 + 1e-5) + be2_ref[...]
    out_ref[...] = jnp.maximum(h, 0.0)


def _make_sc_segsum():
    mesh = plsc.VectorSubcoreMesh(core_axis_name="c", subcore_axis_name="s")

    @functools.partial(
        pl.kernel,
        mesh=mesh,
        out_type=jax.ShapeDtypeStruct((2, NUM_GRAPHS, DIM), jnp.float32),
        scratch_types=[
            pltpu.VMEM((CPW, CHUNK), jnp.int32),
            *[pltpu.VMEM((CHUNK, DIM), jnp.float32) for _ in range(NBUF)],
            pltpu.VMEM_SHARED((ACC_ROWS, DIM), jnp.float32),
            *[pltpu.SemaphoreType.DMA for _ in range(NBUF)],
        ],
    )
    def segsum(emb_hbm, bvw_hbm, zeros_hbm, out_hbm, idx_v, *rest):
        bufs = rest[:NBUF]
        acc_sh = rest[NBUF]
        gsems = rest[NBUF + 1:]

        cid = lax.axis_index("c")
        sid = lax.axis_index("s")
        wid = sid * 2 + cid

        @pl.when(sid == 0)
        def _():
            pltpu.sync_copy(zeros_hbm, acc_sh)

        plsc.subcore_barrier()

        pltpu.sync_copy(bvw_hbm.at[wid], idx_v)

        def chunk_base(c_local):
            return jnp.minimum((wid * CPW + c_local) * CHUNK, LAST_BASE)

        for b in range(NBUF):
            pltpu.async_copy(
                emb_hbm.at[pl.ds(chunk_base(b), CHUNK)], bufs[b], gsems[b])

        def group(g, carry):
            for b in range(NBUF):
                cl = g * NBUF + b
                pltpu.make_async_copy(
                    emb_hbm.at[pl.ds(0, CHUNK)], bufs[b], gsems[b]).wait()
                pltpu.sync_copy(bufs[b], acc_sh.at[idx_v.at[cl]], add=True)

                @pl.when(g < CPW // NBUF - 1)
                def _():
                    pltpu.async_copy(
                        emb_hbm.at[pl.ds(chunk_base(cl + NBUF), CHUNK)],
                        bufs[b], gsems[b])

            return carry

        lax.fori_loop(0, CPW // NBUF, group, 0)
        plsc.subcore_barrier()

        @pl.when(sid == 0)
        def _():
            pltpu.sync_copy(acc_sh.at[pl.ds(0, NUM_GRAPHS)], out_hbm.at[cid])

    return segsum


_sc_segsum = _make_sc_segsum()


@jax.jit
def kernel(virtual_node, embeddings, batch_vector, W1, b1, g1, be1,
           W2, b2, g2, be2):
    bv = batch_vector.astype(jnp.int32)
    pad = jnp.full((96,), NUM_GRAPHS, jnp.int32)
    pad_tail = jnp.full((NUM_WORKERS * CPW * CHUNK - N_NODES - 96,),
                        NUM_GRAPHS, jnp.int32)
    bvw = jnp.concatenate([bv[:LAST_BASE + 96], pad, bv[LAST_BASE + 96:],
                           pad_tail]).reshape(NUM_WORKERS, CPW, CHUNK)
    zeros = jnp.zeros((ACC_ROWS, DIM), jnp.float32)
    partials = _sc_segsum(embeddings, bvw, zeros)

    full = lambda s: pl.BlockSpec(s, lambda: (0,) * len(s))
    out = pl.pallas_call(
        _mlp_body,
        in_specs=[
            full((2, NUM_GRAPHS, DIM)), full((NUM_GRAPHS, DIM)),
            full((2 * DIM, DIM)), full((1, 2 * DIM)), full((1, 2 * DIM)),
            full((1, 2 * DIM)),
            full((DIM, 2 * DIM)), full((1, DIM)), full((1, DIM)),
            full((1, DIM)),
        ],
        out_specs=full((NUM_GRAPHS, DIM)),
        out_shape=jax.ShapeDtypeStruct((NUM_GRAPHS, DIM), jnp.float32),
    )(partials, virtual_node, W1, b1.reshape(1, -1), g1.reshape(1, -1),
      be1.reshape(1, -1), W2, b2.reshape(1, -1), g2.reshape(1, -1),
      be2.reshape(1, -1))
    return out


# speedup vs baseline: 1.9074x; 1.0676x over previous
import functools

import jax
import jax.numpy as jnp
from jax import lax
from jax.experimental import pallas as pl
from jax.experimental.pallas import tpu as pltpu, tpu_sc as plsc

NUM_GRAPHS = 512
N_NODES = 100000
DIM = 128

CHUNK = 128
NUM_WORKERS = 32
CPW = 25
NBUF = 5
FULL_CHUNKS = N_NODES // CHUNK
TAIL = N_NODES - FULL_CHUNKS * CHUNK


def _mlp_body(p_ref, vn_ref, w1_ref, b1_ref, g1_ref, be1_ref,
              w2_ref, b2_ref, g2_ref, be2_ref, out_ref):
    vn = vn_ref[...] + p_ref[0] + p_ref[1]
    h = jax.lax.dot_general(
        vn, w1_ref[...], dimension_numbers=(((1,), (1,)), ((), ())),
        preferred_element_type=jnp.float32,
    ) + b1_ref[...]
    mu = jnp.mean(h, axis=0, keepdims=True)
    var = jnp.mean((h - mu) ** 2, axis=0, keepdims=True)
    h = g1_ref[...] * (h - mu) * jax.lax.rsqrt(var + 1e-5) + be1_ref[...]
    h = jnp.maximum(h, 0.0)
    h = jax.lax.dot_general(
        h, w2_ref[...], dimension_numbers=(((1,), (1,)), ((), ())),
        preferred_element_type=jnp.float32,
    ) + b2_ref[...]
    mu2 = jnp.mean(h, axis=0, keepdims=True)
    var2 = jnp.mean((h - mu2) ** 2, axis=0, keepdims=True)
    h = g2_ref[...] * (h - mu2) * jax.lax.rsqrt(var2 + 1e-5) + be2_ref[...]
    out_ref[...] = jnp.maximum(h, 0.0)


def _make_sc_segsum():
    mesh = plsc.VectorSubcoreMesh(core_axis_name="c", subcore_axis_name="s")

    @functools.partial(
        pl.kernel,
        mesh=mesh,
        out_type=jax.ShapeDtypeStruct((2, NUM_GRAPHS, DIM), jnp.float32),
        scratch_types=[
            pltpu.VMEM((TAIL,), jnp.int32),
            pltpu.VMEM((TAIL, DIM), jnp.float32),
            *[pltpu.VMEM((CHUNK, DIM), jnp.float32) for _ in range(NBUF)],
            *[pltpu.VMEM((CHUNK,), jnp.int32) for _ in range(NBUF)],
            pltpu.VMEM_SHARED((NUM_GRAPHS, DIM), jnp.float32),
            *[pltpu.SemaphoreType.DMA for _ in range(2 * NBUF)],
        ],
    )
    def segsum(emb_hbm, bv_hbm, zeros_hbm, out_hbm,
               tidx_v, trows_v, *rest):
        bufs = rest[:NBUF]
        ibufs = rest[NBUF:2 * NBUF]
        acc_sh = rest[2 * NBUF]
        gsems = rest[2 * NBUF + 1:3 * NBUF + 1]
        isems = rest[3 * NBUF + 1:]

        cid = lax.axis_index("c")
        sid = lax.axis_index("s")
        wid = sid * 2 + cid
        c0 = wid * CPW

        @pl.when(sid == 0)
        def _():
            pltpu.sync_copy(zeros_hbm, acc_sh)

        plsc.subcore_barrier()

        def start_chunk(c, b):
            base = c * CHUNK
            pltpu.async_copy(emb_hbm.at[pl.ds(base, CHUNK)],
                             bufs[b], gsems[b])
            pltpu.async_copy(bv_hbm.at[pl.ds(base, CHUNK)],
                             ibufs[b], isems[b])

        for b in range(NBUF):
            @pl.when(c0 + b < FULL_CHUNKS)
            def _(b=b):
                start_chunk(c0 + b, b)

        def group(g, carry):
            for b in range(NBUF):
                cl = g * NBUF + b

                @pl.when(c0 + cl < FULL_CHUNKS)
                def _(b=b, cl=cl):
                    pltpu.make_async_copy(
                        emb_hbm.at[pl.ds(0, CHUNK)], bufs[b], gsems[b]).wait()
                    pltpu.make_async_copy(
                        bv_hbm.at[pl.ds(0, CHUNK)], ibufs[b],
                        isems[b]).wait()
                    pltpu.sync_copy(bufs[b], acc_sh.at[ibufs[b]], add=True)

                    @pl.when((cl + NBUF < CPW)
                             & (c0 + cl + NBUF < FULL_CHUNKS))
                    def _():
                        start_chunk(c0 + cl + NBUF, b)

            return carry

        lax.fori_loop(0, CPW // NBUF, group, 0)

        @pl.when(wid == NUM_WORKERS - 1)
        def _():
            pltpu.sync_copy(bv_hbm.at[pl.ds(FULL_CHUNKS * CHUNK, TAIL)],
                            tidx_v)
            pltpu.sync_copy(emb_hbm.at[pl.ds(FULL_CHUNKS * CHUNK, TAIL)],
                            trows_v)
            pltpu.sync_copy(trows_v, acc_sh.at[tidx_v], add=True)

        plsc.subcore_barrier()

        @pl.when(sid == 0)
        def _():
            pltpu.sync_copy(acc_sh, out_hbm.at[cid])

    return segsum


_sc_segsum = _make_sc_segsum()


@jax.jit
def kernel(virtual_node, embeddings, batch_vector, W1, b1, g1, be1,
           W2, b2, g2, be2):
    bv = batch_vector.astype(jnp.int32)
    zeros = jnp.zeros((NUM_GRAPHS, DIM), jnp.float32)
    partials = _sc_segsum(embeddings, bv, zeros)

    full = lambda s: pl.BlockSpec(s, lambda: (0,) * len(s))
    out = pl.pallas_call(
        _mlp_body,
        in_specs=[
            full((2, NUM_GRAPHS, DIM)), full((NUM_GRAPHS, DIM)),
            full((2 * DIM, DIM)), full((1, 2 * DIM)), full((1, 2 * DIM)),
            full((1, 2 * DIM)),
            full((DIM, 2 * DIM)), full((1, DIM)), full((1, DIM)),
            full((1, DIM)),
        ],
        out_specs=full((NUM_GRAPHS, DIM)),
        out_shape=jax.ShapeDtypeStruct((NUM_GRAPHS, DIM), jnp.float32),
    )(partials, virtual_node, W1, b1.reshape(1, -1), g1.reshape(1, -1),
      be1.reshape(1, -1), W2, b2.reshape(1, -1), g2.reshape(1, -1),
      be2.reshape(1, -1))
    return out


# speedup vs baseline: 2.1258x; 1.1145x over previous
import functools

import jax
import jax.numpy as jnp
from jax import lax
from jax.experimental import pallas as pl
from jax.experimental.pallas import tpu as pltpu, tpu_sc as plsc

NUM_GRAPHS = 512
N_NODES = 100000
DIM = 128

SC_ROWS = 56000
CHUNK = 112
NUM_SC_CHUNKS = SC_ROWS // CHUNK
NUM_WORKERS = 32
CPW = 16
NBUF = 4

TC_BLOCK = 2000
TC_OFF = SC_ROWS // TC_BLOCK
TC_STEPS = (N_NODES - SC_ROWS) // TC_BLOCK


def _tc_segsum_body(bv_ref, emb_ref, out_ref):
    i = pl.program_id(0)

    @pl.when(i == 0)
    def _():
        out_ref[...] = jnp.zeros_like(out_ref)

    seg = bv_ref[0, 0, :]
    iota = jax.lax.broadcasted_iota(jnp.int32, (NUM_GRAPHS, TC_BLOCK), 0)
    onehot = (seg[None, :] == iota).astype(jnp.float32)
    out_ref[...] += jax.lax.dot_general(
        onehot, emb_ref[...],
        dimension_numbers=(((1,), (0,)), ((), ())),
        preferred_element_type=jnp.float32,
    )


def _mlp_body(p_ref, gtc_ref, vn_ref, w1_ref, b1_ref, g1_ref, be1_ref,
              w2_ref, b2_ref, g2_ref, be2_ref, out_ref):
    vn = vn_ref[...] + gtc_ref[...] + p_ref[0] + p_ref[1]
    h = jax.lax.dot_general(
        vn, w1_ref[...], dimension_numbers=(((1,), (1,)), ((), ())),
        preferred_element_type=jnp.float32,
    ) + b1_ref[...]
    mu = jnp.mean(h, axis=0, keepdims=True)
    var = jnp.mean((h - mu) ** 2, axis=0, keepdims=True)
    h = g1_ref[...] * (h - mu) * jax.lax.rsqrt(var + 1e-5) + be1_ref[...]
    h = jnp.maximum(h, 0.0)
    h = jax.lax.dot_general(
        h, w2_ref[...], dimension_numbers=(((1,), (1,)), ((), ())),
        preferred_element_type=jnp.float32,
    ) + b2_ref[...]
    mu2 = jnp.mean(h, axis=0, keepdims=True)
    var2 = jnp.mean((h - mu2) ** 2, axis=0, keepdims=True)
    h = g2_ref[...] * (h - mu2) * jax.lax.rsqrt(var2 + 1e-5) + be2_ref[...]
    out_ref[...] = jnp.maximum(h, 0.0)


def _make_sc_segsum():
    mesh = plsc.VectorSubcoreMesh(core_axis_name="c", subcore_axis_name="s")

    @functools.partial(
        pl.kernel,
        mesh=mesh,
        out_type=jax.ShapeDtypeStruct((2, NUM_GRAPHS, DIM), jnp.float32),
        scratch_types=[
            *[pltpu.VMEM((CHUNK, DIM), jnp.float32) for _ in range(NBUF)],
            *[pltpu.VMEM((CHUNK,), jnp.int32) for _ in range(NBUF)],
            pltpu.VMEM_SHARED((NUM_GRAPHS, DIM), jnp.float32),
            *[pltpu.SemaphoreType.DMA for _ in range(2 * NBUF)],
        ],
    )
    def segsum(emb_hbm, bv_hbm, zeros_hbm, out_hbm, *rest):
        bufs = rest[:NBUF]
        ibufs = rest[NBUF:2 * NBUF]
        acc_sh = rest[2 * NBUF]
        gsems = rest[2 * NBUF + 1:3 * NBUF + 1]
        isems = rest[3 * NBUF + 1:]

        cid = lax.axis_index("c")
        sid = lax.axis_index("s")
        wid = sid * 2 + cid
        c0 = wid * CPW

        @pl.when(sid == 0)
        def _():
            pltpu.sync_copy(zeros_hbm, acc_sh)

        plsc.subcore_barrier()

        def start_chunk(c, b):
            base = c * CHUNK
            pltpu.async_copy(emb_hbm.at[pl.ds(base, CHUNK)],
                             bufs[b], gsems[b])
            pltpu.async_copy(bv_hbm.at[pl.ds(base, CHUNK)],
                             ibufs[b], isems[b])

        for b in range(NBUF):
            @pl.when(c0 + b < NUM_SC_CHUNKS)
            def _(b=b):
                start_chunk(c0 + b, b)

        def group(g, carry):
            for b in range(NBUF):
                cl = g * NBUF + b

                @pl.when(c0 + cl < NUM_SC_CHUNKS)
                def _(b=b, cl=cl):
                    pltpu.make_async_copy(
                        emb_hbm.at[pl.ds(0, CHUNK)], bufs[b], gsems[b]).wait()
                    pltpu.make_async_copy(
                        bv_hbm.at[pl.ds(0, CHUNK)], ibufs[b],
                        isems[b]).wait()
                    pltpu.sync_copy(bufs[b], acc_sh.at[ibufs[b]], add=True)

                    @pl.when((cl + NBUF < CPW)
                             & (c0 + cl + NBUF < NUM_SC_CHUNKS))
                    def _():
                        start_chunk(c0 + cl + NBUF, b)

            return carry

        lax.fori_loop(0, CPW // NBUF, group, 0)
        plsc.subcore_barrier()

        @pl.when(sid == 0)
        def _():
            pltpu.sync_copy(acc_sh, out_hbm.at[cid])

    return segsum


_sc_segsum = _make_sc_segsum()


@jax.jit
def kernel(virtual_node, embeddings, batch_vector, W1, b1, g1, be1,
           W2, b2, g2, be2):
    bv = batch_vector.astype(jnp.int32)
    zeros = jnp.zeros((NUM_GRAPHS, DIM), jnp.float32)
    partials = _sc_segsum(embeddings, bv, zeros)

    bv3d = bv.reshape(N_NODES // TC_BLOCK, 1, TC_BLOCK)
    g_tc = pl.pallas_call(
        _tc_segsum_body,
        grid=(TC_STEPS,),
        in_specs=[
            pl.BlockSpec((1, 1, TC_BLOCK), lambda i: (i + TC_OFF, 0, 0)),
            pl.BlockSpec((TC_BLOCK, DIM), lambda i: (i + TC_OFF, 0)),
        ],
        out_specs=pl.BlockSpec((NUM_GRAPHS, DIM), lambda i: (0, 0)),
        out_shape=jax.ShapeDtypeStruct((NUM_GRAPHS, DIM), jnp.float32),
    )(bv3d, embeddings)

    full = lambda s: pl.BlockSpec(s, lambda: (0,) * len(s))
    out = pl.pallas_call(
        _mlp_body,
        in_specs=[
            full((2, NUM_GRAPHS, DIM)), full((NUM_GRAPHS, DIM)),
            full((NUM_GRAPHS, DIM)),
            full((2 * DIM, DIM)), full((1, 2 * DIM)), full((1, 2 * DIM)),
            full((1, 2 * DIM)),
            full((DIM, 2 * DIM)), full((1, DIM)), full((1, DIM)),
            full((1, DIM)),
        ],
        out_specs=full((NUM_GRAPHS, DIM)),
        out_shape=jax.ShapeDtypeStruct((NUM_GRAPHS, DIM), jnp.float32),
    )(partials, g_tc, virtual_node, W1, b1.reshape(1, -1), g1.reshape(1, -1),
      be1.reshape(1, -1), W2, b2.reshape(1, -1), g2.reshape(1, -1),
      be2.reshape(1, -1))
    return out
